# speedup vs baseline: 9.3389x; 9.3389x over previous
import jax
import jax.numpy as jnp
from jax.experimental import pallas as pl

_K = 64
_ROWS_PER_BLOCK = 16


def _topk_mask_block(x_ref, o_ref):
    x = x_ref[...]
    b = jax.lax.bitcast_convert_type(x, jnp.uint32)
    u = jnp.where(b >> 31 == 1, ~b, b | jnp.uint32(0x80000000))

    t = jnp.zeros((x.shape[0], 1), dtype=jnp.uint32)
    for bit in range(31, -1, -1):
        cand = t | jnp.uint32(1 << bit)
        cnt = jnp.sum((u >= cand).astype(jnp.int32), axis=1, keepdims=True)
        t = jnp.where(cnt >= _K, cand, t)

    o_ref[...] = jnp.where(u >= t, x, jnp.float32(0.0))


def kernel(x):
    rows, cols = x.shape
    grid = rows // _ROWS_PER_BLOCK
    return pl.pallas_call(
        _topk_mask_block,
        grid=(grid,),
        in_specs=[pl.BlockSpec((_ROWS_PER_BLOCK, cols), lambda i: (i, 0))],
        out_specs=pl.BlockSpec((_ROWS_PER_BLOCK, cols), lambda i: (i, 0)),
        out_shape=jax.ShapeDtypeStruct(x.shape, x.dtype),
    )(x)


# speedup vs baseline: 13.8067x; 1.4784x over previous
import jax
import jax.numpy as jnp
from jax.experimental import pallas as pl
from jax.experimental.pallas import tpu as pltpu

_K = 64
_R = 16
_N = 32768
_NW = 64
_W = _N // _NW
_MAXIT = 24
def _as_int(ub):
    return jax.lax.bitcast_convert_type(ub ^ jnp.uint32(0x80000000), jnp.int32)


def _topk_mask_block(x_ref, o_ref, s_ref):
    x = x_ref[...]
    bi = jax.lax.bitcast_convert_type(x, jnp.int32)
    s = jnp.where(bi >= 0, bi, jnp.int32(-2147483648) - bi)
    s_ref[...] = s

    m = s[:, 0:_W]
    for k in range(1, _NW):
        m = jnp.maximum(m, s[:, k * _W:(k + 1) * _W])

    lo = jnp.zeros((_R, 1), jnp.uint32)
    for bit in range(31, -1, -1):
        cand = lo | jnp.uint32(1 << bit)
        cnt = jnp.sum((m >= _as_int(cand)).astype(jnp.int32), axis=1,
                      keepdims=True)
        lo = jnp.where(cnt >= _K, cand, lo)
    mx = jnp.max(m, axis=1, keepdims=True)
    hi = (jax.lax.bitcast_convert_type(mx, jnp.uint32)
          ^ jnp.uint32(0x80000000)) + jnp.uint32(1)
    clo = jnp.sum((s >= _as_int(lo)).astype(jnp.int32), axis=1, keepdims=True)
    chi = jnp.zeros((_R, 1), jnp.int32)

    def _open(lo, hi, clo):
        d = jax.lax.bitcast_convert_type(hi - lo, jnp.int32)
        return (d != 0) & (d != 1) & (clo != _K)

    def cond(st):
        i, lo, hi, clo, chi = st
        return (i < _MAXIT) & jnp.any(_open(lo, hi, clo))

    def body(st):
        i, lo, hi, clo, chi = st
        is_open = _open(lo, hi, clo)
        wdiff = jax.lax.bitcast_convert_type(hi - lo, jnp.int32)
        width = jnp.where(wdiff < 0, jnp.float32(2.1e9),
                          wdiff.astype(jnp.float32))
        llo = jnp.log2(clo.astype(jnp.float32))
        lhi = jnp.log2(jnp.maximum(chi.astype(jnp.float32), 0.5))
        frac = (llo - 6.0) / jnp.maximum(llo - lhi, 1e-6)
        off = jnp.where((i & 1) == 1, width * 0.5, frac * width)
        off = jnp.clip(off, 1.0, jnp.maximum(width - 1.0, 1.0))
        cand = lo + jax.lax.bitcast_convert_type(off.astype(jnp.int32),
                                                 jnp.uint32)
        lo1 = lo + jnp.uint32(1)
        hi1 = hi - jnp.uint32(1)
        cand = jnp.where(_as_int(cand) < _as_int(lo1), lo1, cand)
        cand = jnp.where(_as_int(cand) > _as_int(hi1), hi1, cand)
        c = jnp.sum((s_ref[...] >= _as_int(cand)).astype(jnp.int32), axis=1,
                    keepdims=True)
        ge = is_open & (c >= _K)
        lt = is_open & (c < _K)
        lo = jnp.where(ge, cand, lo)
        clo = jnp.where(ge, c, clo)
        hi = jnp.where(lt, cand, hi)
        chi = jnp.where(lt, c, chi)
        return i + 1, lo, hi, clo, chi

    _, lo, hi, clo, chi = jax.lax.while_loop(
        cond, body, (jnp.int32(0), lo, hi, clo, chi))

    o_ref[...] = jnp.where(s_ref[...] >= _as_int(lo), x, jnp.float32(0.0))


def kernel(x):
    rows, cols = x.shape
    grid = rows // _R
    return pl.pallas_call(
        _topk_mask_block,
        grid=(grid,),
        in_specs=[pl.BlockSpec((_R, cols), lambda i: (i, 0))],
        out_specs=pl.BlockSpec((_R, cols), lambda i: (i, 0)),
        out_shape=jax.ShapeDtypeStruct(x.shape, x.dtype),
        scratch_shapes=[pltpu.VMEM((_R, _N), jnp.int32)],
    )(x)


# speedup vs baseline: 17.9118x; 1.2973x over previous
import jax
import jax.numpy as jnp
from jax.experimental import pallas as pl

_K = 64
_R = 32
_N = 32768
_W = 512
_NS = _N // _W
_MAXIT = 16


def _enc(f):
    bi = jax.lax.bitcast_convert_type(f, jnp.int32)
    return jnp.where(bi >= 0, bi, jnp.int32(-2147483648) - bi)


def _dec(e):
    bi = jnp.where(e >= 0, e, jnp.int32(-2147483648) - e)
    return jax.lax.bitcast_convert_type(bi, jnp.float32)


def _pass(x_ref, cand):
    inf = jnp.float32(jnp.inf)
    xs = x_ref[:, 0:_W]
    km = xs >= cand
    acc_c = km.astype(jnp.int32)
    acc_mn = jnp.where(km, xs, inf)
    acc_mx = jnp.where(km, -inf, xs)
    for k in range(1, _NS):
        xs = x_ref[:, k * _W:(k + 1) * _W]
        km = xs >= cand
        acc_c = acc_c + km.astype(jnp.int32)
        acc_mn = jnp.minimum(acc_mn, jnp.where(km, xs, inf))
        acc_mx = jnp.maximum(acc_mx, jnp.where(km, -inf, xs))
    c = jnp.sum(acc_c, axis=1, keepdims=True)
    smin = jnp.min(acc_mn, axis=1, keepdims=True)
    mlt = jnp.max(acc_mx, axis=1, keepdims=True)
    return c, smin, mlt


def _topk_mask_block(x_ref, o_ref):
    m = x_ref[:, 0:_W]
    for k in range(1, _NS):
        m = jnp.maximum(m, x_ref[:, k * _W:(k + 1) * _W])
    g = m[:, 0:64]
    for k in range(1, 8):
        g = jnp.maximum(g, m[:, k * 64:(k + 1) * 64])
    lo0 = jnp.min(g, axis=1, keepdims=True)
    hi = _dec(_enc(jnp.max(g, axis=1, keepdims=True)) + 1)

    c0, smin0, _ = _pass(x_ref, lo0)
    lo = smin0
    clo = c0
    l1 = jnp.log2(c0.astype(jnp.float32))
    v1 = smin0
    v0 = hi
    l0 = jnp.full((_R, 1), -1.0, dtype=jnp.float32)

    def _open(lo, hi, clo):
        return (_enc(hi) - _enc(lo) > 1) & (clo != _K)

    def cond(st):
        i, lo, hi, clo, v1, l1, v0, l0 = st
        return (i < _MAXIT) & jnp.any(_open(lo, hi, clo))

    def body(st):
        i, lo, hi, clo, v1, l1, v0, l0 = st
        is_open = _open(lo, hi, clo)
        el, eh = _enc(lo), _enc(hi)
        denom = l0 - l1
        degen = (jnp.abs(denom) < 1e-6) | (v0 == v1)
        cand_sec = v1 + (6.0 - l1) * (v0 - v1) / jnp.where(degen, 1.0, denom)
        ce = jnp.where(degen, el + (eh - el) // 2, _enc(cand_sec))
        ce = jnp.minimum(jnp.maximum(ce, el + 1), eh - 1)
        cand = _dec(ce)

        c, smin, mlt = _pass(x_ref, cand)
        lc = jnp.log2(jnp.maximum(c.astype(jnp.float32), 0.5))
        ge = is_open & (c >= _K)
        lt = is_open & (c < _K)
        lo = jnp.where(ge, smin, lo)
        clo = jnp.where(ge, c, clo)
        hi = jnp.where(lt, _dec(_enc(mlt) + 1), hi)
        newv = jnp.where(ge, smin, mlt)
        newl = jnp.where(ge, lc,
                         jnp.log2((c + 1).astype(jnp.float32)))
        v0 = jnp.where(is_open, v1, v0)
        l0 = jnp.where(is_open, l1, l0)
        v1 = jnp.where(is_open, newv, v1)
        l1 = jnp.where(is_open, newl, l1)
        return i + 1, lo, hi, clo, v1, l1, v0, l0

    _, lo, hi, clo, v1, l1, v0, l0 = jax.lax.while_loop(
        cond, body, (jnp.int32(0), lo, hi, clo, v1, l1, v0, l0))

    x = x_ref[...]
    o_ref[...] = jnp.where(x >= lo, x, jnp.float32(0.0))


def kernel(x):
    rows, cols = x.shape
    grid = rows // _R
    return pl.pallas_call(
        _topk_mask_block,
        grid=(grid,),
        in_specs=[pl.BlockSpec((_R, cols), lambda i: (i, 0))],
        out_specs=pl.BlockSpec((_R, cols), lambda i: (i, 0)),
        out_shape=jax.ShapeDtypeStruct(x.shape, x.dtype),
    )(x)
